# baseline (speedup 1.0000x reference)
import math

import jax
import jax.numpy as jnp
from jax.experimental import pallas as pl
from jax.experimental.pallas import tpu as pltpu

_TOPK = 32
_MAX_BISECT = 13


def _attn_body(q_ref, k_ref, v_ref, o_ref):
    T = q_ref.shape[1]
    E = q_ref.shape[2] // 2
    S = k_ref.shape[1]
    D = v_ref.shape[2] // 2
    temp = 1.0 / math.sqrt(E)

    q = q_ref[0] * temp
    k = k_ref[0]
    se = jax.lax.dot_general(
        q[:, :E], k[:, :E], (((1,), (1,)), ((), ())),
        preferred_element_type=jnp.float32,
    )
    so = jax.lax.dot_general(
        q[:, E:], k[:, E:], (((1,), (1,)), ((), ())),
        preferred_element_type=jnp.float32,
    )
    scores = jnp.concatenate([se, so], axis=0)

    m = jnp.max(scores, axis=1, keepdims=True)
    kf32 = jnp.float32(_TOPK)

    lo0 = jnp.min(scores[:, :_TOPK], axis=1, keepdims=True)
    hi0 = m + jnp.float32(1.0)

    def bisect_body(_, carry):
        lo, hi = carry
        d = hi - lo
        m1 = lo + 0.25 * d
        m2 = lo + 0.5 * d
        m3 = lo + 0.75 * d

        def count(p):
            msk = jnp.where(scores >= p, jnp.float32(1.0), jnp.float32(0.0))
            return jnp.sum(msk, axis=1, keepdims=True)

        g1 = count(m1) >= kf32
        g2 = count(m2) >= kf32
        g3 = count(m3) >= kf32
        lo = jnp.where(g3, m3, jnp.where(g2, m2, jnp.where(g1, m1, lo)))
        hi = jnp.where(g3, hi, jnp.where(g2, m3, jnp.where(g1, m2, m1)))
        return lo, hi

    t, _ = jax.lax.fori_loop(0, _MAX_BISECT, bisect_body, (lo0, hi0))

    num = jnp.where(scores >= t, jnp.exp(scores - m), jnp.float32(0.0))
    den = jnp.sum(num, axis=1, keepdims=True)
    attn = num * (1.0 / den)

    v = v_ref[0]
    oe = jax.lax.dot_general(
        attn[:T], v[:, :D], (((1,), (0,)), ((), ())),
        preferred_element_type=jnp.float32,
    )
    oo = jax.lax.dot_general(
        attn[T:], v[:, D:], (((1,), (0,)), ((), ())),
        preferred_element_type=jnp.float32,
    )
    o_ref[0] = jnp.concatenate([oe, oo], axis=1)


def kernel(query, key, value):
    B, T, H, E = query.shape
    S = key.shape[1]
    D = value.shape[3]

    qf = query.reshape(B, T, H * E)
    kf = key.reshape(B, S, H * E)
    vf = value.reshape(B, S, H * D)

    grid = (B, H // 2)
    out = pl.pallas_call(
        _attn_body,
        grid=grid,
        in_specs=[
            pl.BlockSpec((1, T, 2 * E), lambda b, hp: (b, 0, hp)),
            pl.BlockSpec((1, S, 2 * E), lambda b, hp: (b, 0, hp)),
            pl.BlockSpec((1, S, 2 * D), lambda b, hp: (b, 0, hp)),
        ],
        out_specs=pl.BlockSpec((1, T, 2 * D), lambda b, hp: (b, 0, hp)),
        out_shape=jax.ShapeDtypeStruct((B, T, H * D), jnp.float32),
        compiler_params=pltpu.CompilerParams(
            dimension_semantics=("parallel", "parallel"),
        ),
    )(qf, kf, vf)
    return out.reshape(B, T, H, D)
